# baseline (speedup 1.0000x reference)
import jax
import jax.numpy as jnp
from jax.experimental import pallas as pl
from jax.experimental.pallas import tpu as pltpu


def kernel(x, timesteps, table):
    B, L, D = x.shape
    bb = 64
    nsteps = B // bb
    max_norm = float(D) ** 0.5

    def body(ts_ref, x_ref, table_ref, o_ref, rows_buf, sems):
        i = pl.program_id(0)

        def issue(step, slot):
            base = step * bb
            for j in range(bb):
                pltpu.make_async_copy(
                    table_ref.at[pl.ds(ts_ref[base + j], 1)],
                    rows_buf.at[slot, pl.ds(j, 1)],
                    sems.at[slot],
                ).start()

        def drain(slot):
            for _ in range(bb):
                pltpu.make_async_copy(
                    table_ref.at[pl.ds(0, 1)],
                    rows_buf.at[slot, pl.ds(0, 1)],
                    sems.at[slot],
                ).wait()

        @pl.when(i == 0)
        def _():
            issue(0, 0)

        @pl.when(i + 1 < nsteps)
        def _():
            issue(i + 1, (i + 1) % 2)

        drain(i % 2)
        e = rows_buf[i % 2]
        norm = jnp.sqrt(jnp.sum(e * e, axis=-1, keepdims=True))
        scale = jnp.where(norm > max_norm, max_norm / (norm + 1e-7),
                          jnp.float32(1.0))
        o_ref[...] = x_ref[...] + (e * scale)[:, None, :]

    grid_spec = pltpu.PrefetchScalarGridSpec(
        num_scalar_prefetch=1,
        grid=(nsteps,),
        in_specs=[
            pl.BlockSpec((bb, L, D), lambda i, ts: (i, 0, 0)),
            pl.BlockSpec(memory_space=pl.ANY),
        ],
        out_specs=pl.BlockSpec((bb, L, D), lambda i, ts: (i, 0, 0)),
        scratch_shapes=[
            pltpu.VMEM((2, bb, D), jnp.float32),
            pltpu.SemaphoreType.DMA((2,)),
        ],
    )
    return pl.pallas_call(
        body,
        grid_spec=grid_spec,
        out_shape=jax.ShapeDtypeStruct((B, L, D), x.dtype),
    )(timesteps.astype(jnp.int32), x, table)


# speedup vs baseline: 1.0176x; 1.0176x over previous
import jax
import jax.numpy as jnp
from jax.experimental import pallas as pl
from jax.experimental.pallas import tpu as pltpu


def kernel(x, timesteps, table):
    B, L, D = x.shape
    bb = 128
    nsteps = B // bb
    max_norm = float(D) ** 0.5

    def body(ts_ref, x_ref, table_ref, o_ref, rows_buf, sems):
        i = pl.program_id(0)

        def issue(step, slot):
            base = step * bb
            for j in range(bb):
                pltpu.make_async_copy(
                    table_ref.at[pl.ds(ts_ref[base + j], 1)],
                    rows_buf.at[slot, pl.ds(j, 1)],
                    sems.at[slot],
                ).start()

        def drain(slot):
            pltpu.make_async_copy(
                table_ref.at[pl.ds(0, bb)],
                rows_buf.at[slot],
                sems.at[slot],
            ).wait()

        @pl.when(i == 0)
        def _():
            issue(0, 0)

        @pl.when(i + 1 < nsteps)
        def _():
            issue(i + 1, (i + 1) % 2)

        drain(i % 2)
        e = rows_buf[i % 2]
        norm = jnp.sqrt(jnp.sum(e * e, axis=-1, keepdims=True))
        scale = jnp.where(norm > max_norm, max_norm / (norm + 1e-7),
                          jnp.float32(1.0))
        o_ref[...] = x_ref[...] + (e * scale)[:, None, :]

    grid_spec = pltpu.PrefetchScalarGridSpec(
        num_scalar_prefetch=1,
        grid=(nsteps,),
        in_specs=[
            pl.BlockSpec((bb, L, D), lambda i, ts: (i, 0, 0)),
            pl.BlockSpec(memory_space=pl.ANY),
        ],
        out_specs=pl.BlockSpec((bb, L, D), lambda i, ts: (i, 0, 0)),
        scratch_shapes=[
            pltpu.VMEM((2, bb, D), jnp.float32),
            pltpu.SemaphoreType.DMA((2,)),
        ],
    )
    return pl.pallas_call(
        body,
        grid_spec=grid_spec,
        out_shape=jax.ShapeDtypeStruct((B, L, D), x.dtype),
    )(timesteps.astype(jnp.int32), x, table)


# speedup vs baseline: 1.0387x; 1.0207x over previous
import jax
import jax.numpy as jnp
from jax import lax
from jax.experimental import pallas as pl
from jax.experimental.pallas import tpu as pltpu


def kernel(x, timesteps, table):
    B, L, D = x.shape
    CB = 32
    NC = B // CB
    NB = 3
    max_norm = float(D) ** 0.5

    def body(ts_ref, x_hbm, table_hbm, o_hbm, xin, xout, rows, isem, osem,
             rsem):
        def start_in(g, slot):
            pltpu.make_async_copy(
                x_hbm.at[pl.ds(g * CB, CB)], xin.at[slot], isem.at[slot]
            ).start()
            base = g * CB
            for j in range(CB):
                pltpu.make_async_copy(
                    table_hbm.at[pl.ds(ts_ref[base + j], 1)],
                    rows.at[slot, pl.ds(j, 1)],
                    rsem.at[slot],
                ).start()

        for s in range(NB):
            start_in(s, s)

        def step(g, _):
            slot = lax.rem(g, NB)
            pltpu.make_async_copy(
                x_hbm.at[pl.ds(0, CB)], xin.at[slot], isem.at[slot]
            ).wait()
            pltpu.make_async_copy(
                table_hbm.at[pl.ds(0, CB)], rows.at[slot], rsem.at[slot]
            ).wait()

            @pl.when(g >= NB)
            def _():
                pltpu.make_async_copy(
                    x_hbm.at[pl.ds(0, CB)], xout.at[slot], osem.at[slot]
                ).wait()

            e = rows[slot]
            norm = jnp.sqrt(jnp.sum(e * e, axis=-1, keepdims=True))
            scale = jnp.where(norm > max_norm, max_norm / (norm + 1e-7),
                              jnp.float32(1.0))
            xout[slot] = xin[slot] + (e * scale)[:, None, :]
            pltpu.make_async_copy(
                xout.at[slot], o_hbm.at[pl.ds(g * CB, CB)], osem.at[slot]
            ).start()

            @pl.when(g + NB < NC)
            def _():
                start_in(g + NB, slot)

            return None

        lax.fori_loop(0, NC, step, None, unroll=False)

        for s in range(NB):
            pltpu.make_async_copy(
                x_hbm.at[pl.ds(0, CB)], xout.at[s], osem.at[s]
            ).wait()

    grid_spec = pltpu.PrefetchScalarGridSpec(
        num_scalar_prefetch=1,
        grid=(1,),
        in_specs=[
            pl.BlockSpec(memory_space=pl.ANY),
            pl.BlockSpec(memory_space=pl.ANY),
        ],
        out_specs=pl.BlockSpec(memory_space=pl.ANY),
        scratch_shapes=[
            pltpu.VMEM((NB, CB, L, D), jnp.float32),
            pltpu.VMEM((NB, CB, L, D), jnp.float32),
            pltpu.VMEM((NB, CB, D), jnp.float32),
            pltpu.SemaphoreType.DMA((NB,)),
            pltpu.SemaphoreType.DMA((NB,)),
            pltpu.SemaphoreType.DMA((NB,)),
        ],
    )
    return pl.pallas_call(
        body,
        grid_spec=grid_spec,
        out_shape=jax.ShapeDtypeStruct((B, L, D), x.dtype),
    )(timesteps.astype(jnp.int32), x, table)
